# baseline (speedup 1.0000x reference)
import functools
import jax
import jax.numpy as jnp
from jax import lax
from jax.experimental import pallas as pl
from jax.experimental.pallas import tpu as pltpu
from jax.experimental.pallas import tpu_sc as plsc

_D = 64
_SEQ = 200
_NC = 2
_NS = 16
_NW = _NC * _NS
_SL = 100
_CHUNK = 400
_SPC = _CHUNK // _SL


def _sc_body(nchunks, table_hbm, idx_hbm, pe_hbm, out_hbm,
             idx_v, pe_v, rows_v, sem):
    wid = lax.axis_index("s") * _NC + lax.axis_index("c")
    idx_rows = nchunks * _SPC
    base_row = wid * idx_rows

    pltpu.sync_copy(idx_hbm.at[pl.ds(base_row * 1, idx_rows)], idx_v)
    pltpu.sync_copy(pe_hbm, pe_v)

    def chunk_body(c, _):
        copies = []
        for j in range(_SPC):
            copies.append(pltpu.async_copy(
                table_hbm.at[idx_v.at[c * _SPC + j]],
                rows_v.at[pl.ds(j * _SL, _SL)],
                sem))
        for cp in copies:
            cp.wait()

        def row_body(i, _):
            for r in range(_CHUNK // _SEQ):
                row = r * _SEQ + i
                for s in range(_D // 16):
                    sl = pl.ds(s * 16, 16)
                    rows_v[row, sl] = rows_v[row, sl] * 8.0 + pe_v[i, sl]
            return ()
        lax.fori_loop(0, _SEQ, row_body, (), unroll=2)

        out_row0 = base_row * _SL + c * _CHUNK
        pltpu.sync_copy(rows_v, out_hbm.at[pl.ds(out_row0, _CHUNK)])
        return ()

    lax.fori_loop(0, nchunks, chunk_body, ())


def kernel(x, table, pe):
    b, seq = x.shape
    n = b * seq
    assert seq == _SEQ and n % (_NW * _CHUNK) == 0
    nchunks = n // (_NW * _CHUNK)
    idx2 = x.reshape(n // _SL, _SL).astype(jnp.int32)
    pe2 = pe[0, :seq, :]

    mesh = plsc.VectorSubcoreMesh(core_axis_name="c", subcore_axis_name="s",
                                  num_cores=_NC, num_subcores=_NS)
    grid_kernel = pl.kernel(
        functools.partial(_sc_body, nchunks),
        out_type=jax.ShapeDtypeStruct((n, _D), jnp.float32),
        mesh=mesh,
        scratch_types=[
            pltpu.VMEM((nchunks * _SPC, _SL), jnp.int32),
            pltpu.VMEM((_SEQ, _D), jnp.float32),
            pltpu.VMEM((_CHUNK, _D), jnp.float32),
            pltpu.SemaphoreType.DMA,
        ],
        compiler_params=pltpu.CompilerParams(use_tc_tiling_on_sc=False),
    )
    out = grid_kernel(table, idx2, pe2)
    return out.reshape(b, seq, _D)


# speedup vs baseline: 1.3908x; 1.3908x over previous
import functools
import jax
import jax.numpy as jnp
from jax import lax
from jax.experimental import pallas as pl
from jax.experimental.pallas import tpu as pltpu
from jax.experimental.pallas import tpu_sc as plsc

_D = 64
_SEQ = 200
_NC = 2
_NS = 16
_NW = _NC * _NS
_SL = 100
_CHUNK = 400
_SPC = _CHUNK // _SL


def _sc_body(nchunks, table_hbm, idx_hbm, pe_hbm, out_hbm,
             idx_v, pe_v, rows0, rows1, gsem0, gsem1):
    wid = lax.axis_index("s") * _NC + lax.axis_index("c")
    idx_rows = nchunks * _SPC
    base_row = wid * idx_rows

    pltpu.sync_copy(idx_hbm.at[pl.ds(base_row, idx_rows)], idx_v)
    pltpu.sync_copy(pe_hbm, pe_v)

    def fire(c, rows_b, gsem):
        for j in range(_SPC):
            pltpu.async_copy(table_hbm.at[idx_v.at[c * _SPC + j]],
                             rows_b.at[pl.ds(j * _SL, _SL)], gsem)

    def wait_gather(rows_b, gsem):
        pltpu.make_async_copy(table_hbm.at[pl.ds(0, _CHUNK)], rows_b,
                              gsem).wait()

    def compute(rows_b):
        @plsc.parallel_loop(0, _SEQ, unroll=4)
        def _(i):
            pvec = [pe_v[i, pl.ds(s * 16, 16)] for s in range(_D // 16)]
            for r in range(_CHUNK // _SEQ):
                row = r * _SEQ + i
                for s in range(_D // 16):
                    sl = pl.ds(s * 16, 16)
                    rows_b[row, sl] = rows_b[row, sl] * 8.0 + pvec[s]

    def store(c, rows_b):
        pltpu.sync_copy(rows_b,
                        out_hbm.at[pl.ds(base_row * _SL + c * _CHUNK, _CHUNK)])

    fire(0, rows0, gsem0)

    def loop_body(t, _):
        c = 2 * t
        fire(c + 1, rows1, gsem1)
        wait_gather(rows0, gsem0)
        compute(rows0)
        store(c, rows0)

        @pl.when(c + 2 < nchunks)
        def _():
            fire(c + 2, rows0, gsem0)
        wait_gather(rows1, gsem1)
        compute(rows1)
        store(c + 1, rows1)
        return ()

    lax.fori_loop(0, nchunks // 2, loop_body, ())


def kernel(x, table, pe):
    b, seq = x.shape
    n = b * seq
    assert seq == _SEQ and n % (_NW * _CHUNK) == 0
    nchunks = n // (_NW * _CHUNK)
    idx2 = x.reshape(n // _SL, _SL).astype(jnp.int32)
    pe2 = pe[0, :seq, :]

    mesh = plsc.VectorSubcoreMesh(core_axis_name="c", subcore_axis_name="s",
                                  num_cores=_NC, num_subcores=_NS)
    grid_kernel = pl.kernel(
        functools.partial(_sc_body, nchunks),
        out_type=jax.ShapeDtypeStruct((n, _D), jnp.float32),
        mesh=mesh,
        scratch_types=[
            pltpu.VMEM((nchunks * _SPC, _SL), jnp.int32),
            pltpu.VMEM((_SEQ, _D), jnp.float32),
            pltpu.VMEM((_CHUNK, _D), jnp.float32),
            pltpu.VMEM((_CHUNK, _D), jnp.float32),
            pltpu.SemaphoreType.DMA,
            pltpu.SemaphoreType.DMA,
        ],
        compiler_params=pltpu.CompilerParams(use_tc_tiling_on_sc=False),
    )
    out = grid_kernel(table, idx2, pe2)
    return out.reshape(b, seq, _D)
